# Initial kernel scaffold; baseline (speedup 1.0000x reference)
import functools

import jax
import jax.numpy as jnp
from jax import lax
from jax.experimental import pallas as pl
from jax.experimental.pallas import tpu as pltpu
from jax.experimental.pallas import tpu_sc as plsc

N = 10000
E = 320000
D = 128
D_OUT = 40

NTILE = 16
NCORE = 2
NW = NTILE * NCORE
NP = 10240
ROWS_PER_TILE = NP // NTILE
K = 128
NCHUNK = 80
EPW = NCHUNK * K
EP = NW * EPW

_mesh = plsc.VectorSubcoreMesh(core_axis_name="c", subcore_axis_name="s")


@functools.partial(
    pl.kernel,
    out_type=(
        jax.ShapeDtypeStruct((NCORE, NP, 16), jnp.float32),
        jax.ShapeDtypeStruct((NCORE, NP, 16), jnp.float32),
    ),
    mesh=_mesh,
    scratch_types=[
        pltpu.VMEM((NCHUNK, K), jnp.int32),
        pltpu.VMEM((NCHUNK, K), jnp.int32),
        pltpu.VMEM((K, 16), jnp.float32),
        pltpu.VMEM((K, 16), jnp.float32),
        pltpu.VMEM_SHARED((NP, 16), jnp.float32),
        pltpu.VMEM_SHARED((NP, 16), jnp.float32),
    ],
)
def _sc_degrees(srcp, dstp, hs_out, hd_out,
                idx_s, idx_d, obuf, zbuf, hist_s, hist_d):
    cid = lax.axis_index("c")
    sid = lax.axis_index("s")
    wid = cid * NTILE + sid

    def fill(i, _):
        obuf[i] = jnp.ones((16,), jnp.float32)
        zbuf[i] = jnp.zeros((16,), jnp.float32)
        return 0
    lax.fori_loop(0, K, fill, 0)

    base = sid * ROWS_PER_TILE
    for j in range(ROWS_PER_TILE // K):
        sl = pl.ds(base + j * K, K)
        pltpu.sync_copy(zbuf, hist_s.at[sl])
        pltpu.sync_copy(zbuf, hist_d.at[sl])
    plsc.subcore_barrier()

    pltpu.sync_copy(srcp.at[wid], idx_s)
    pltpu.sync_copy(dstp.at[wid], idx_d)

    def chunk(j, _):
        pltpu.sync_copy(obuf, hist_s.at[idx_s.at[j]], add=True)
        pltpu.sync_copy(obuf, hist_d.at[idx_d.at[j]], add=True)
        return 0
    lax.fori_loop(0, NCHUNK, chunk, 0)
    plsc.subcore_barrier()

    for j in range(ROWS_PER_TILE // K):
        sl = pl.ds(base + j * K, K)
        pltpu.sync_copy(hist_s.at[sl], hs_out.at[cid, sl])
        pltpu.sync_copy(hist_d.at[sl], hd_out.at[cid, sl])


@functools.partial(
    pl.kernel,
    out_type=jax.ShapeDtypeStruct((NCORE, NP, D), jnp.float32),
    mesh=_mesh,
    scratch_types=[
        pltpu.VMEM((NCHUNK, K), jnp.int32),
        pltpu.VMEM((NCHUNK, K), jnp.int32),
        pltpu.VMEM((K, D), jnp.float32),
        pltpu.VMEM((K, D), jnp.float32),
        pltpu.VMEM((K, D), jnp.float32),
        pltpu.VMEM_SHARED((NP, D), jnp.float32),
        pltpu.SemaphoreType.DMA,
        pltpu.SemaphoreType.DMA,
    ],
)
def _sc_aggregate(xn, srcp, dstp, pout,
                  idx_s, idx_d, rows0, rows1, zbuf, agg, sem0, sem1):
    cid = lax.axis_index("c")
    sid = lax.axis_index("s")
    wid = cid * NTILE + sid

    def fill(i, _):
        r = i // (D // 16)
        c = (i % (D // 16)) * 16
        zbuf[r, pl.ds(c, 16)] = jnp.zeros((16,), jnp.float32)
        return 0
    lax.fori_loop(0, K * (D // 16), fill, 0)

    base = sid * ROWS_PER_TILE
    for j in range(ROWS_PER_TILE // K):
        pltpu.sync_copy(zbuf, agg.at[pl.ds(base + j * K, K)])
    plsc.subcore_barrier()

    pltpu.sync_copy(srcp.at[wid], idx_s)
    pltpu.sync_copy(dstp.at[wid], idx_d)

    rows = (rows0, rows1)
    sems = (sem0, sem1)
    for b in range(2):
        pltpu.async_copy(xn.at[idx_s.at[b]], rows[b], sems[b])

    def outer(g, _):
        for b in range(2):
            j = g * 2 + b
            pltpu.make_async_copy(xn.at[idx_s.at[j]], rows[b], sems[b]).wait()
            pltpu.sync_copy(rows[b], agg.at[idx_d.at[j]], add=True)

            @pl.when(j + 2 < NCHUNK)
            def _():
                pltpu.async_copy(xn.at[idx_s.at[j + 2]], rows[b], sems[b])
        return 0
    lax.fori_loop(0, NCHUNK // 2, outer, 0)
    plsc.subcore_barrier()

    for j in range(ROWS_PER_TILE // K):
        sl = pl.ds(base + j * K, K)
        pltpu.sync_copy(agg.at[sl], pout.at[cid, sl])


BLK = 256


def _tc_norms_body(hs_ref, hd_ref, h_ref, xn_ref, nin_ref, nout_ref):
    i = pl.program_id(0)
    deg_out = hs_ref[0][:, :1] + hs_ref[1][:, :1]
    deg_in = hd_ref[0][:, :1] + hd_ref[1][:, :1]
    no = lax.rsqrt(jnp.maximum(deg_out, 1.0))
    ni = lax.rsqrt(jnp.maximum(deg_in, 1.0))
    row = i * BLK + lax.broadcasted_iota(jnp.int32, (BLK, 1), 0)
    valid = row < N
    no = jnp.where(valid, no, 0.0)
    ni = jnp.where(valid, ni, 0.0)
    nout_ref[...] = jnp.broadcast_to(no, (BLK, D))
    nin_ref[...] = jnp.broadcast_to(ni, (BLK, D))
    xn_ref[...] = h_ref[...] * no


def _tc_norms(hs, hd, h_pad):
    return pl.pallas_call(
        _tc_norms_body,
        grid=(NP // BLK,),
        in_specs=[
            pl.BlockSpec((NCORE, BLK, 16), lambda i: (0, i, 0)),
            pl.BlockSpec((NCORE, BLK, 16), lambda i: (0, i, 0)),
            pl.BlockSpec((BLK, D), lambda i: (i, 0)),
        ],
        out_specs=[
            pl.BlockSpec((BLK, D), lambda i: (i, 0)),
            pl.BlockSpec((BLK, D), lambda i: (i, 0)),
            pl.BlockSpec((BLK, D), lambda i: (i, 0)),
        ],
        out_shape=[
            jax.ShapeDtypeStruct((NP, D), jnp.float32),
            jax.ShapeDtypeStruct((NP, D), jnp.float32),
            jax.ShapeDtypeStruct((NP, D), jnp.float32),
        ],
    )(hs, hd, h_pad)


def _tc_layer1_body(p_ref, nin_ref, nout_ref, w_ref, b_ref, o_ref):
    acc = (p_ref[0] + p_ref[1]) * nin_ref[...]
    y = jnp.dot(acc, w_ref[...], preferred_element_type=jnp.float32)
    y = jnp.maximum(y + b_ref[...], 0.0)
    o_ref[...] = y * nout_ref[...]


def _tc_layer1(p, nin, nout, W, b):
    return pl.pallas_call(
        _tc_layer1_body,
        grid=(NP // BLK,),
        in_specs=[
            pl.BlockSpec((NCORE, BLK, D), lambda i: (0, i, 0)),
            pl.BlockSpec((BLK, D), lambda i: (i, 0)),
            pl.BlockSpec((BLK, D), lambda i: (i, 0)),
            pl.BlockSpec((D, D), lambda i: (0, 0)),
            pl.BlockSpec((1, D), lambda i: (0, 0)),
        ],
        out_specs=pl.BlockSpec((BLK, D), lambda i: (i, 0)),
        out_shape=jax.ShapeDtypeStruct((NP, D), jnp.float32),
    )(p, nin, nout, W, b)


def _tc_layer2_body(p_ref, nin_ref, w_ref, b_ref, o_ref):
    i = pl.program_id(0)
    acc = (p_ref[0] + p_ref[1]) * nin_ref[...]
    y = jnp.dot(acc, w_ref[...], preferred_element_type=jnp.float32)
    y = jnp.maximum(y + b_ref[...], 0.0)
    row = i * BLK + lax.broadcasted_iota(jnp.int32, (BLK, 1), 0)
    y = jnp.where(row < N, y, 0.0)

    @pl.when(i == 0)
    def _():
        o_ref[...] = jnp.zeros_like(o_ref)

    o_ref[...] += jnp.sum(y, axis=0, keepdims=True)


def _tc_layer2(p, nin, W, b):
    return pl.pallas_call(
        _tc_layer2_body,
        grid=(NP // BLK,),
        in_specs=[
            pl.BlockSpec((NCORE, BLK, D), lambda i: (0, i, 0)),
            pl.BlockSpec((BLK, D), lambda i: (i, 0)),
            pl.BlockSpec((D, D), lambda i: (0, 0)),
            pl.BlockSpec((1, D), lambda i: (0, 0)),
        ],
        out_specs=pl.BlockSpec((1, D), lambda i: (0, 0)),
        out_shape=jax.ShapeDtypeStruct((1, D), jnp.float32),
    )(p, nin, W, b)


def _tc_head_body(cs_ref, wm_ref, bm_ref, o_ref):
    hg = cs_ref[...] * (1.0 / N)
    logits = jnp.dot(hg, wm_ref[...], preferred_element_type=jnp.float32)
    logits = logits + bm_ref[...]
    m = jnp.max(logits, axis=0, keepdims=True)
    e = jnp.exp(logits - m)
    lse = m + jnp.log(jnp.sum(e, axis=0, keepdims=True))
    o_ref[...] = logits - lse


def _tc_head(cs, Wm, bm):
    return pl.pallas_call(
        _tc_head_body,
        out_shape=jax.ShapeDtypeStruct((1, D_OUT), jnp.float32),
    )(cs, Wm, bm)


def kernel(h, edge_index, W1, b1, W2, b2, Wm, bm):
    src = edge_index[0].astype(jnp.int32)
    dst = edge_index[1].astype(jnp.int32)
    pad = EP - E
    srcp = jnp.concatenate([src, jnp.full((pad,), N, jnp.int32)])
    dstp = jnp.concatenate([dst, jnp.full((pad,), N, jnp.int32)])
    srcp = srcp.reshape(NW, NCHUNK, K)
    dstp = dstp.reshape(NW, NCHUNK, K)

    hs, hd = _sc_degrees(srcp, dstp)
    h_pad = jnp.pad(h, ((0, NP - N), (0, 0)))
    xn1, nin, nout = _tc_norms(hs, hd, h_pad)
    p1 = _sc_aggregate(xn1, srcp, dstp)
    xn2 = _tc_layer1(p1, nin, nout, W1, b1.reshape(1, D))
    p2 = _sc_aggregate(xn2, srcp, dstp)
    cs = _tc_layer2(p2, nin, W2, b2.reshape(1, D))
    return _tc_head(cs, Wm, bm.reshape(1, D_OUT))


# speedup vs baseline: 3.0944x; 3.0944x over previous
import functools

import jax
import jax.numpy as jnp
from jax import lax
from jax.experimental import pallas as pl
from jax.experimental.pallas import tpu as pltpu
from jax.experimental.pallas import tpu_sc as plsc

N = 10000
E = 320000
D = 128
D_OUT = 40

NTILE = 16
NCORE = 2
NW = NTILE * NCORE
NP = 10240
ROWS_PER_TILE = NP // NTILE
K = 128
NCHUNK = 80
EPW = NCHUNK * K
EP = NW * EPW

_mesh = plsc.VectorSubcoreMesh(core_axis_name="c", subcore_axis_name="s")


@functools.partial(
    pl.kernel,
    out_type=(
        jax.ShapeDtypeStruct((NW * NP,), jnp.float32),
        jax.ShapeDtypeStruct((NW * NP,), jnp.float32),
    ),
    mesh=_mesh,
    scratch_types=[
        pltpu.VMEM((EPW,), jnp.int32),
        pltpu.VMEM((EPW,), jnp.int32),
        pltpu.VMEM((NP,), jnp.float32),
        pltpu.VMEM((NP,), jnp.float32),
    ],
    compiler_params=pltpu.CompilerParams(needs_layout_passes=False),
)
def _sc_degrees(srcf, dstf, hs_out, hd_out,
                idxs, idxd, hist_s, hist_d):
    cid = lax.axis_index("c")
    sid = lax.axis_index("s")
    wid = cid * NTILE + sid
    ebase = wid * EPW

    pltpu.sync_copy(srcf.at[pl.ds(ebase, EPW)], idxs)
    pltpu.sync_copy(dstf.at[pl.ds(ebase, EPW)], idxd)

    def fillz(i, _):
        hist_s[pl.ds(i * 16, 16)] = jnp.zeros((16,), jnp.float32)
        hist_d[pl.ds(i * 16, 16)] = jnp.zeros((16,), jnp.float32)
        return 0
    lax.fori_loop(0, NP // 16, fillz, 0)

    ones16 = jnp.ones((16,), jnp.float32)

    def acc(i, _):
        i16 = idxs[pl.ds(i * 16, 16)]
        plsc.addupdate_scatter(hist_s, [i16], ones16)
        j16 = idxd[pl.ds(i * 16, 16)]
        plsc.addupdate_scatter(hist_d, [j16], ones16)
        return 0
    lax.fori_loop(0, EPW // 16, acc, 0)

    pltpu.sync_copy(hist_s, hs_out.at[pl.ds(wid * NP, NP)])
    pltpu.sync_copy(hist_d, hd_out.at[pl.ds(wid * NP, NP)])


GK = 64
GCH = EPW // GK
NBUF = 4

@functools.partial(
    pl.kernel,
    out_type=jax.ShapeDtypeStruct((NCORE, NP, D), jnp.float32),
    mesh=_mesh,
    scratch_types=[
        pltpu.VMEM((EPW,), jnp.int32),
        pltpu.VMEM((GK,), jnp.int32),
        pltpu.VMEM((GK,), jnp.int32),
        pltpu.VMEM((GK,), jnp.int32),
        pltpu.VMEM((GK,), jnp.int32),
        pltpu.VMEM((GK, D), jnp.float32),
        pltpu.VMEM((GK, D), jnp.float32),
        pltpu.VMEM((GK, D), jnp.float32),
        pltpu.VMEM((GK, D), jnp.float32),
        pltpu.VMEM_SHARED((NP, D), jnp.float32),
        pltpu.SemaphoreType.DMA,
        pltpu.SemaphoreType.DMA,
        pltpu.SemaphoreType.DMA,
        pltpu.SemaphoreType.DMA,
        pltpu.SemaphoreType.DMA,
        pltpu.SemaphoreType.DMA,
        pltpu.SemaphoreType.DMA,
        pltpu.SemaphoreType.DMA,
    ],
)
def _sc_aggregate(xn, srcf, dstf, pout,
                  idxs, db0, db1, db2, db3, r0, r1, r2, r3, agg,
                  ds0, ds1, ds2, ds3, gs0, gs1, gs2, gs3):
    cid = lax.axis_index("c")
    sid = lax.axis_index("s")
    wid = cid * NTILE + sid
    dbufs = (db0, db1, db2, db3)
    dsems = (ds0, ds1, ds2, ds3)
    rows = (r0, r1, r2, r3)
    gsems = (gs0, gs1, gs2, gs3)

    def fill(i, _):
        r = i // (D // 16)
        c = (i % (D // 16)) * 16
        r0[r, pl.ds(c, 16)] = jnp.zeros((16,), jnp.float32)
        return 0
    lax.fori_loop(0, GK * (D // 16), fill, 0)

    base = sid * ROWS_PER_TILE
    for j in range(ROWS_PER_TILE // GK):
        pltpu.sync_copy(r0, agg.at[pl.ds(base + j * GK, GK)])
    plsc.subcore_barrier()

    ebase = wid * EPW
    pltpu.sync_copy(srcf.at[pl.ds(ebase, EPW)], idxs)
    for j in range(NBUF):
        pltpu.async_copy(dstf.at[pl.ds(ebase + j * GK, GK)], dbufs[j], dsems[j])
        pltpu.async_copy(xn.at[idxs.at[pl.ds(j * GK, GK)]], rows[j], gsems[j])

    def outer(g, _):
        for q in range(NBUF):
            j = g * NBUF + q
            pltpu.make_async_copy(
                xn.at[idxs.at[pl.ds(j * GK, GK)]], rows[q], gsems[q]).wait()
            pltpu.make_async_copy(
                dstf.at[pl.ds(ebase + j * GK, GK)], dbufs[q], dsems[q]).wait()
            pltpu.sync_copy(rows[q], agg.at[dbufs[q]], add=True)

            @pl.when(j + NBUF < GCH)
            def _():
                pltpu.async_copy(
                    dstf.at[pl.ds(ebase + (j + NBUF) * GK, GK)],
                    dbufs[q], dsems[q])
                pltpu.async_copy(
                    xn.at[idxs.at[pl.ds((j + NBUF) * GK, GK)]],
                    rows[q], gsems[q])
        return 0
    lax.fori_loop(0, GCH // NBUF, outer, 0)
    plsc.subcore_barrier()

    for j in range(ROWS_PER_TILE // GK):
        sl = pl.ds(base + j * GK, GK)
        pltpu.sync_copy(agg.at[sl], r0)
        pltpu.sync_copy(r0, pout.at[cid, sl])


BLK = 256


def _tc_norms_body(hs_ref, hd_ref, h_ref, xn_ref, nin_ref, nout_ref):
    i = pl.program_id(0)
    onesw = jnp.ones((NW, 1), jnp.float32)
    cdims = (((0,), (0,)), ((), ()))
    deg_out = lax.dot_general(hs_ref[...], onesw, cdims,
                              preferred_element_type=jnp.float32)
    deg_in = lax.dot_general(hd_ref[...], onesw, cdims,
                             preferred_element_type=jnp.float32)
    no = lax.rsqrt(jnp.maximum(deg_out, 1.0))
    ni = lax.rsqrt(jnp.maximum(deg_in, 1.0))
    row = i * BLK + lax.broadcasted_iota(jnp.int32, (BLK, 1), 0)
    valid = row < N
    no = jnp.where(valid, no, 0.0)
    ni = jnp.where(valid, ni, 0.0)
    nout_ref[...] = jnp.broadcast_to(no, (BLK, D))
    nin_ref[...] = jnp.broadcast_to(ni, (BLK, D))
    xn_ref[...] = h_ref[...] * no


def _tc_norms(hs, hd, h_pad):
    return pl.pallas_call(
        _tc_norms_body,
        grid=(NP // BLK,),
        in_specs=[
            pl.BlockSpec((NW, BLK), lambda i: (0, i)),
            pl.BlockSpec((NW, BLK), lambda i: (0, i)),
            pl.BlockSpec((BLK, D), lambda i: (i, 0)),
        ],
        out_specs=[
            pl.BlockSpec((BLK, D), lambda i: (i, 0)),
            pl.BlockSpec((BLK, D), lambda i: (i, 0)),
            pl.BlockSpec((BLK, D), lambda i: (i, 0)),
        ],
        out_shape=[
            jax.ShapeDtypeStruct((NP, D), jnp.float32),
            jax.ShapeDtypeStruct((NP, D), jnp.float32),
            jax.ShapeDtypeStruct((NP, D), jnp.float32),
        ],
    )(hs, hd, h_pad)


def _tc_layer1_body(p_ref, nin_ref, nout_ref, w_ref, b_ref, o_ref):
    acc = (p_ref[0] + p_ref[1]) * nin_ref[...]
    y = jnp.dot(acc, w_ref[...], preferred_element_type=jnp.float32)
    y = jnp.maximum(y + b_ref[...], 0.0)
    o_ref[...] = y * nout_ref[...]


def _tc_layer1(p, nin, nout, W, b):
    return pl.pallas_call(
        _tc_layer1_body,
        grid=(NP // BLK,),
        in_specs=[
            pl.BlockSpec((NCORE, BLK, D), lambda i: (0, i, 0)),
            pl.BlockSpec((BLK, D), lambda i: (i, 0)),
            pl.BlockSpec((BLK, D), lambda i: (i, 0)),
            pl.BlockSpec((D, D), lambda i: (0, 0)),
            pl.BlockSpec((1, D), lambda i: (0, 0)),
        ],
        out_specs=pl.BlockSpec((BLK, D), lambda i: (i, 0)),
        out_shape=jax.ShapeDtypeStruct((NP, D), jnp.float32),
    )(p, nin, nout, W, b)


def _tc_layer2_body(p_ref, nin_ref, w_ref, b_ref, o_ref):
    i = pl.program_id(0)
    acc = (p_ref[0] + p_ref[1]) * nin_ref[...]
    y = jnp.dot(acc, w_ref[...], preferred_element_type=jnp.float32)
    y = jnp.maximum(y + b_ref[...], 0.0)
    row = i * BLK + lax.broadcasted_iota(jnp.int32, (BLK, 1), 0)
    y = jnp.where(row < N, y, 0.0)

    @pl.when(i == 0)
    def _():
        o_ref[...] = jnp.zeros_like(o_ref)

    o_ref[...] += jnp.sum(y, axis=0, keepdims=True)


def _tc_layer2(p, nin, W, b):
    return pl.pallas_call(
        _tc_layer2_body,
        grid=(NP // BLK,),
        in_specs=[
            pl.BlockSpec((NCORE, BLK, D), lambda i: (0, i, 0)),
            pl.BlockSpec((BLK, D), lambda i: (i, 0)),
            pl.BlockSpec((D, D), lambda i: (0, 0)),
            pl.BlockSpec((1, D), lambda i: (0, 0)),
        ],
        out_specs=pl.BlockSpec((1, D), lambda i: (0, 0)),
        out_shape=jax.ShapeDtypeStruct((1, D), jnp.float32),
    )(p, nin, W, b)


def _tc_head_body(cs_ref, wm_ref, bm_ref, o_ref):
    hg = cs_ref[...] * (1.0 / N)
    logits = jnp.dot(hg, wm_ref[...], preferred_element_type=jnp.float32)
    logits = logits + bm_ref[...]
    m = jnp.max(logits, axis=0, keepdims=True)
    e = jnp.exp(logits - m)
    lse = m + jnp.log(jnp.sum(e, axis=0, keepdims=True))
    o_ref[...] = logits - lse


def _tc_head(cs, Wm, bm):
    return pl.pallas_call(
        _tc_head_body,
        out_shape=jax.ShapeDtypeStruct((1, D_OUT), jnp.float32),
    )(cs, Wm, bm)


def kernel(h, edge_index, W1, b1, W2, b2, Wm, bm):
    pad = EP - E
    ei = jnp.pad(edge_index.astype(jnp.int32), ((0, 0), (0, pad)),
                 constant_values=N)
    srcf = ei[0]
    dstf = ei[1]

    hs, hd = _sc_degrees(srcf, dstf)
    hs = hs.reshape(NW, NP)
    hd = hd.reshape(NW, NP)
    h_pad = jnp.pad(h, ((0, NP - N), (0, 0)))
    xn1, nin, nout = _tc_norms(hs, hd, h_pad)
    p1 = _sc_aggregate(xn1, srcf, dstf)
    xn2 = _tc_layer1(p1, nin, nout, W1, b1.reshape(1, D))
    p2 = _sc_aggregate(xn2, srcf, dstf)
    cs = _tc_layer2(p2, nin, W2, b2.reshape(1, D))
    return _tc_head(cs, Wm, bm.reshape(1, D_OUT))


# speedup vs baseline: 3.0977x; 1.0011x over previous
import functools

import jax
import jax.numpy as jnp
from jax import lax
from jax.experimental import pallas as pl
from jax.experimental.pallas import tpu as pltpu
from jax.experimental.pallas import tpu_sc as plsc

N = 10000
E = 320000
D = 128
D_OUT = 40

NTILE = 16
NCORE = 2
NW = NTILE * NCORE
NP = 10240
ROWS_PER_TILE = NP // NTILE
K = 128
NCHUNK = 80
EPW = NCHUNK * K
EP = NW * EPW

_mesh = plsc.VectorSubcoreMesh(core_axis_name="c", subcore_axis_name="s")


@functools.partial(
    pl.kernel,
    out_type=(
        jax.ShapeDtypeStruct((NW * NP,), jnp.float32),
        jax.ShapeDtypeStruct((NW * NP,), jnp.float32),
    ),
    mesh=_mesh,
    scratch_types=[
        pltpu.VMEM((EPW,), jnp.int32),
        pltpu.VMEM((EPW,), jnp.int32),
        pltpu.VMEM((NP,), jnp.float32),
        pltpu.VMEM((NP,), jnp.float32),
    ],
    compiler_params=pltpu.CompilerParams(needs_layout_passes=False),
)
def _sc_degrees(srcf, dstf, hs_out, hd_out,
                idxs, idxd, hist_s, hist_d):
    cid = lax.axis_index("c")
    sid = lax.axis_index("s")
    wid = cid * NTILE + sid
    ebase = wid * EPW

    pltpu.sync_copy(srcf.at[pl.ds(ebase, EPW)], idxs)
    pltpu.sync_copy(dstf.at[pl.ds(ebase, EPW)], idxd)

    def fillz(i, _):
        hist_s[pl.ds(i * 16, 16)] = jnp.zeros((16,), jnp.float32)
        hist_d[pl.ds(i * 16, 16)] = jnp.zeros((16,), jnp.float32)
        return 0
    lax.fori_loop(0, NP // 16, fillz, 0)

    ones16 = jnp.ones((16,), jnp.float32)

    def acc(i, _):
        i16 = idxs[pl.ds(i * 16, 16)]
        plsc.addupdate_scatter(hist_s, [i16], ones16)
        j16 = idxd[pl.ds(i * 16, 16)]
        plsc.addupdate_scatter(hist_d, [j16], ones16)
        return 0
    lax.fori_loop(0, EPW // 16, acc, 0)

    pltpu.sync_copy(hist_s, hs_out.at[pl.ds(wid * NP, NP)])
    pltpu.sync_copy(hist_d, hd_out.at[pl.ds(wid * NP, NP)])


@functools.partial(
    pl.kernel,
    out_type=jax.ShapeDtypeStruct((NCORE, NP, D), jnp.float32),
    mesh=_mesh,
    scratch_types=[
        pltpu.VMEM((EPW,), jnp.int32),
        pltpu.VMEM((K,), jnp.int32),
        pltpu.VMEM((K,), jnp.int32),
        pltpu.VMEM((K,), jnp.int32),
        pltpu.VMEM((K,), jnp.int32),
        pltpu.VMEM((K, D), jnp.float32),
        pltpu.VMEM((K, D), jnp.float32),
        pltpu.VMEM_SHARED((NP, D), jnp.float32),
        pltpu.SemaphoreType.DMA,
        pltpu.SemaphoreType.DMA,
        pltpu.SemaphoreType.DMA,
        pltpu.SemaphoreType.DMA,
        pltpu.SemaphoreType.DMA,
        pltpu.SemaphoreType.DMA,
    ],
)
def _sc_aggregate(xn, srcf, dstf, pout,
                  idxs, db0, db1, db2, db3, rows0, rows1, agg,
                  ds0, ds1, ds2, ds3, gs0, gs1):
    cid = lax.axis_index("c")
    sid = lax.axis_index("s")
    wid = cid * NTILE + sid
    dbufs = (db0, db1, db2, db3)
    dsems = (ds0, ds1, ds2, ds3)
    rows = (rows0, rows1)
    gsems = (gs0, gs1)

    def fill(i, _):
        r = i // (D // 16)
        c = (i % (D // 16)) * 16
        rows0[r, pl.ds(c, 16)] = jnp.zeros((16,), jnp.float32)
        return 0
    lax.fori_loop(0, K * (D // 16), fill, 0)

    base = sid * ROWS_PER_TILE
    for j in range(ROWS_PER_TILE // K):
        pltpu.sync_copy(rows0, agg.at[pl.ds(base + j * K, K)])
    plsc.subcore_barrier()

    ebase = wid * EPW
    pltpu.sync_copy(srcf.at[pl.ds(ebase, EPW)], idxs)
    for j in range(4):
        pltpu.async_copy(dstf.at[pl.ds(ebase + j * K, K)], dbufs[j], dsems[j])
    for j in range(2):
        pltpu.async_copy(xn.at[idxs.at[pl.ds(j * K, K)]], rows[j], gsems[j])

    def outer(g, _):
        for q in range(4):
            j = g * 4 + q
            b = q % 2
            pltpu.make_async_copy(
                xn.at[idxs.at[pl.ds(j * K, K)]], rows[b], gsems[b]).wait()
            pltpu.make_async_copy(
                dstf.at[pl.ds(ebase + j * K, K)], dbufs[q], dsems[q]).wait()
            pltpu.sync_copy(rows[b], agg.at[dbufs[q]], add=True)

            @pl.when(j + 4 < NCHUNK)
            def _():
                pltpu.async_copy(
                    dstf.at[pl.ds(ebase + (j + 4) * K, K)], dbufs[q], dsems[q])

            @pl.when(j + 2 < NCHUNK)
            def _():
                pltpu.async_copy(
                    xn.at[idxs.at[pl.ds((j + 2) * K, K)]], rows[b], gsems[b])
        return 0
    lax.fori_loop(0, NCHUNK // 4, outer, 0)
    plsc.subcore_barrier()

    for j in range(ROWS_PER_TILE // K):
        sl = pl.ds(base + j * K, K)
        pltpu.sync_copy(agg.at[sl], rows0)
        pltpu.sync_copy(rows0, pout.at[cid, sl])


BLK = 256


def _tc_norms_body(hs_ref, hd_ref, h_ref, xn_ref, nin_ref, nout_ref):
    i = pl.program_id(0)
    onesw = jnp.ones((NW, 1), jnp.float32)
    cdims = (((0,), (0,)), ((), ()))
    deg_out = lax.dot_general(hs_ref[...], onesw, cdims,
                              preferred_element_type=jnp.float32)
    deg_in = lax.dot_general(hd_ref[...], onesw, cdims,
                             preferred_element_type=jnp.float32)
    no = lax.rsqrt(jnp.maximum(deg_out, 1.0))
    ni = lax.rsqrt(jnp.maximum(deg_in, 1.0))
    row = i * BLK + lax.broadcasted_iota(jnp.int32, (BLK, 1), 0)
    valid = row < N
    no = jnp.where(valid, no, 0.0)
    ni = jnp.where(valid, ni, 0.0)
    nout_ref[...] = jnp.broadcast_to(no, (BLK, D))
    nin_ref[...] = jnp.broadcast_to(ni, (BLK, D))
    xn_ref[...] = h_ref[...] * no


def _tc_norms(hs, hd, h_pad):
    return pl.pallas_call(
        _tc_norms_body,
        grid=(NP // BLK,),
        in_specs=[
            pl.BlockSpec((NW, BLK), lambda i: (0, i)),
            pl.BlockSpec((NW, BLK), lambda i: (0, i)),
            pl.BlockSpec((BLK, D), lambda i: (i, 0)),
        ],
        out_specs=[
            pl.BlockSpec((BLK, D), lambda i: (i, 0)),
            pl.BlockSpec((BLK, D), lambda i: (i, 0)),
            pl.BlockSpec((BLK, D), lambda i: (i, 0)),
        ],
        out_shape=[
            jax.ShapeDtypeStruct((NP, D), jnp.float32),
            jax.ShapeDtypeStruct((NP, D), jnp.float32),
            jax.ShapeDtypeStruct((NP, D), jnp.float32),
        ],
    )(hs, hd, h_pad)


def _tc_layer1_body(p_ref, nin_ref, nout_ref, w_ref, b_ref, o_ref):
    acc = (p_ref[0] + p_ref[1]) * nin_ref[...]
    y = jnp.dot(acc, w_ref[...], preferred_element_type=jnp.float32)
    y = jnp.maximum(y + b_ref[...], 0.0)
    o_ref[...] = y * nout_ref[...]


def _tc_layer1(p, nin, nout, W, b):
    return pl.pallas_call(
        _tc_layer1_body,
        grid=(NP // BLK,),
        in_specs=[
            pl.BlockSpec((NCORE, BLK, D), lambda i: (0, i, 0)),
            pl.BlockSpec((BLK, D), lambda i: (i, 0)),
            pl.BlockSpec((BLK, D), lambda i: (i, 0)),
            pl.BlockSpec((D, D), lambda i: (0, 0)),
            pl.BlockSpec((1, D), lambda i: (0, 0)),
        ],
        out_specs=pl.BlockSpec((BLK, D), lambda i: (i, 0)),
        out_shape=jax.ShapeDtypeStruct((NP, D), jnp.float32),
    )(p, nin, nout, W, b)


def _tc_layer2_body(p_ref, nin_ref, w_ref, b_ref, o_ref):
    i = pl.program_id(0)
    acc = (p_ref[0] + p_ref[1]) * nin_ref[...]
    y = jnp.dot(acc, w_ref[...], preferred_element_type=jnp.float32)
    y = jnp.maximum(y + b_ref[...], 0.0)
    row = i * BLK + lax.broadcasted_iota(jnp.int32, (BLK, 1), 0)
    y = jnp.where(row < N, y, 0.0)

    @pl.when(i == 0)
    def _():
        o_ref[...] = jnp.zeros_like(o_ref)

    o_ref[...] += jnp.sum(y, axis=0, keepdims=True)


def _tc_layer2(p, nin, W, b):
    return pl.pallas_call(
        _tc_layer2_body,
        grid=(NP // BLK,),
        in_specs=[
            pl.BlockSpec((NCORE, BLK, D), lambda i: (0, i, 0)),
            pl.BlockSpec((BLK, D), lambda i: (i, 0)),
            pl.BlockSpec((D, D), lambda i: (0, 0)),
            pl.BlockSpec((1, D), lambda i: (0, 0)),
        ],
        out_specs=pl.BlockSpec((1, D), lambda i: (0, 0)),
        out_shape=jax.ShapeDtypeStruct((1, D), jnp.float32),
    )(p, nin, W, b)


def _tc_head_body(cs_ref, wm_ref, bm_ref, o_ref):
    hg = cs_ref[...] * (1.0 / N)
    logits = jnp.dot(hg, wm_ref[...], preferred_element_type=jnp.float32)
    logits = logits + bm_ref[...]
    m = jnp.max(logits, axis=0, keepdims=True)
    e = jnp.exp(logits - m)
    lse = m + jnp.log(jnp.sum(e, axis=0, keepdims=True))
    o_ref[...] = logits - lse


def _tc_head(cs, Wm, bm):
    return pl.pallas_call(
        _tc_head_body,
        out_shape=jax.ShapeDtypeStruct((1, D_OUT), jnp.float32),
    )(cs, Wm, bm)


def kernel(h, edge_index, W1, b1, W2, b2, Wm, bm):
    pad = EP - E
    ei = jnp.pad(edge_index.astype(jnp.int32), ((0, 0), (0, pad)),
                 constant_values=N)
    srcf = ei[0]
    dstf = ei[1]

    hs, hd = _sc_degrees(srcf, dstf)
    hs = hs.reshape(NW, NP)
    hd = hd.reshape(NW, NP)
    h_pad = jnp.pad(h, ((0, NP - N), (0, 0)))
    xn1, nin, nout = _tc_norms(hs, hd, h_pad)
    p1 = _sc_aggregate(xn1, srcf, dstf)
    xn2 = _tc_layer1(p1, nin, nout, W1, b1.reshape(1, D))
    p2 = _sc_aggregate(xn2, srcf, dstf)
    cs = _tc_layer2(p2, nin, W2, b2.reshape(1, D))
    return _tc_head(cs, Wm, bm.reshape(1, D_OUT))
